# baseline (speedup 1.0000x reference)
import functools

import jax
import jax.numpy as jnp
from jax import lax
from jax.experimental import pallas as pl
from jax.experimental.pallas import tpu as pltpu
from jax.experimental.pallas import tpu_sc as plsc

B, N, K, SX, F, MID, DOUT, C, SO = 4, 10000, 16, 12, 32, 16, 16, 8, 32
MARGIN = 0.5
UPDATE_RATE = 0.01

R = B * N
CSO = C * SO
W = CSO + 2 * C
NCORES, NSUB = 2, 16
NW = NCORES * NSUB
CH = 128
NCH = 10
PER_TILE = CH * NCH
RP = NW * PER_TILE
BLK = 2000
NBLK = R // BLK


def _relu(x):
    return jnp.maximum(x, 0.0)


def _dot(a, b):
    return jnp.dot(a, b, preferred_element_type=jnp.float32)


def _expand_mat(rows, cols, group):
    r = lax.broadcasted_iota(jnp.int32, (rows, cols), 0)
    j = lax.broadcasted_iota(jnp.int32, (rows, cols), 1)
    return (j // group == r).astype(jnp.float32)


def _s1_body(fushed_ref, inp_ref, cent_ref, wc1a, bc1a, wc1b, bc1b, wc2, bc2,
             wi1a, bi1a, wi1b, bi1b, wi2, bi2, wgt, bgt, out_ref):
    x = fushed_ref[...]
    fused = _relu(_dot(_relu(_dot(x, wi1a[...]) + bi1a[...]), wi1b[...])
                  + bi1b[...]) + _relu(_dot(x, wi2[...]) + bi2[...])
    cent = cent_ref[...]
    cf = _relu(_dot(_relu(_dot(cent, wc1a[...]) + bc1a[...]), wc1b[...])
               + bc1b[...]) + _relu(_dot(cent, wc2[...]) + bc2[...])
    xn = jnp.sum(fused * fused, axis=1, keepdims=True)
    cn = lax.dot_general(jnp.ones((1, DOUT), jnp.float32), cf * cf,
                         (((1,), (1,)), ((), ())))
    g = lax.dot_general(fused, cf, (((1,), (1,)), ((), ())))
    dist = jnp.sqrt(jnp.maximum(xn + cn - 2.0 * g, 1e-30))
    m = jnp.max(dist, axis=1, keepdims=True)
    e = jnp.exp(dist - m)
    simi = e / jnp.sum(e, axis=1, keepdims=True)
    t = _relu(_dot(inp_ref[...], wgt[...]) + bgt[...])
    h = t * _dot(simi, _expand_mat(C, CSO, SO))
    out_ref[...] = jnp.concatenate(
        [h, simi, jnp.zeros((BLK, C), jnp.float32)], axis=1)


def _stage1(fushed, inp, cent, wc1a, bc1a, wc1b, bc1b, wc2, bc2,
            wi1a, bi1a, wi1b, bi1b, wi2, bi2, wgt, bgt):
    row_spec = lambda width: pl.BlockSpec((BLK, width), lambda i: (i, 0))
    rep = lambda shape: pl.BlockSpec(shape, lambda i: (0,) * len(shape))
    return pl.pallas_call(
        _s1_body,
        grid=(NBLK,),
        in_specs=[row_spec(F), row_spec(SX), rep((C, F)),
                  rep((F, MID)), rep((1, MID)), rep((MID, DOUT)), rep((1, DOUT)),
                  rep((F, DOUT)), rep((1, DOUT)),
                  rep((F, MID)), rep((1, MID)), rep((MID, DOUT)), rep((1, DOUT)),
                  rep((F, DOUT)), rep((1, DOUT)),
                  rep((SX, CSO)), rep((1, CSO))],
        out_specs=row_spec(W),
        out_shape=jax.ShapeDtypeStruct((R, W), jnp.float32),
    )(fushed, inp, cent, wc1a, bc1a, wc1b, bc1b, wc2, bc2,
      wi1a, bi1a, wi1b, bi1b, wi2, bi2, wgt, bgt)


def _s2_body(hs_hbm, adjt_hbm, out_hbm, idx_v, acc_v, acc2_v, sem0, sem1):
    wid = lax.axis_index("s") * NCORES + lax.axis_index("c")
    pltpu.sync_copy(adjt_hbm.at[:, wid], idx_v)
    half = K // 2

    def chunk(j, carry):
        descs = [pltpu.async_copy(hs_hbm.at[idx_v.at[0, j]], acc_v, sem0),
                 pltpu.async_copy(hs_hbm.at[idx_v.at[half, j]], acc2_v, sem0)]
        descs += [pltpu.async_copy(hs_hbm.at[idx_v.at[k, j]], acc_v, sem1,
                                   add=True) for k in range(1, half)]
        descs += [pltpu.async_copy(hs_hbm.at[idx_v.at[k, j]], acc2_v, sem1,
                                   add=True) for k in range(half + 1, K)]
        for d in descs:
            d.wait()

        def reduce_row(r, carry2):
            for v in range(W // 16):
                sl = pl.ds(v * 16, 16)
                acc_v[r, sl] = acc_v[r, sl] + acc2_v[r, sl]
            return carry2

        lax.fori_loop(0, CH, reduce_row, 0)
        pltpu.sync_copy(acc_v, out_hbm.at[pl.ds(wid * PER_TILE + j * CH, CH)])
        return carry

    lax.fori_loop(0, NCH, chunk, 0)


_stage2 = functools.partial(
    pl.kernel,
    out_type=jax.ShapeDtypeStruct((RP, W), jnp.float32),
    mesh=plsc.VectorSubcoreMesh(core_axis_name="c", subcore_axis_name="s",
                                num_cores=NCORES, num_subcores=NSUB),
    scratch_types=[pltpu.VMEM((K, NCH, CH), jnp.int32),
                   pltpu.VMEM((CH, W), jnp.float32),
                   pltpu.VMEM((CH, W), jnp.float32),
                   pltpu.SemaphoreType.DMA,
                   pltpu.SemaphoreType.DMA],
    compiler_params=pltpu.CompilerParams(use_tc_tiling_on_sc=False),
)(_s2_body)


def _s3_body(nd_ref, upd_ref, sum_ref):
    nd = nd_ref[...]
    numer = nd[:, :CSO]
    denom = nd[:, CSO:CSO + C]
    upd = numer / _dot(denom, _expand_mat(C, CSO, SO))
    upd_ref[...] = upd
    part = jnp.sum(upd, axis=0, keepdims=True)

    @pl.when(pl.program_id(0) == 0)
    def _():
        sum_ref[...] = part

    @pl.when(pl.program_id(0) > 0)
    def _():
        sum_ref[...] = sum_ref[...] + part


def _stage3(nd):
    return pl.pallas_call(
        _s3_body,
        grid=(NBLK,),
        in_specs=[pl.BlockSpec((BLK, W), lambda i: (i, 0))],
        out_specs=[pl.BlockSpec((BLK, CSO), lambda i: (i, 0)),
                   pl.BlockSpec((1, CSO), lambda i: (0, 0))],
        out_shape=[jax.ShapeDtypeStruct((R, CSO), jnp.float32),
                   jax.ShapeDtypeStruct((1, CSO), jnp.float32)],
    )(nd)


def _s4_body(sum_ref, cent_ref, out_ref):
    mean = sum_ref[...] * (1.0 / R)
    nc = (1.0 - UPDATE_RATE) * cent_ref[...] + UPDATE_RATE * mean
    sq = nc * nc
    ones = jnp.ones((1, SO), jnp.float32)
    ncol = lax.dot_general(ones, sq, (((1,), (1,)), ((), ())))
    nrow = lax.dot_general(sq, ones, (((1,), (1,)), ((), ())))
    g = lax.dot_general(nc, nc, (((1,), (1,)), ((), ())))
    dist = jnp.sqrt(jnp.maximum(nrow + ncol - 2.0 * g, 1e-30))
    i = lax.broadcasted_iota(jnp.int32, (C, C), 0)
    j = lax.broadcasted_iota(jnp.int32, (C, C), 1)
    target = jnp.where(i == j, 0.0, MARGIN)
    out_ref[...] = jnp.sum(jnp.maximum(target - dist, 0.0) ** 2,
                           keepdims=True)


def _stage4(sums, cent):
    return pl.pallas_call(
        _s4_body,
        in_specs=[pl.BlockSpec((C, SO), lambda: (0, 0)),
                  pl.BlockSpec((C, SO), lambda: (0, 0))],
        out_specs=pl.BlockSpec((1, 1), lambda: (0, 0)),
        out_shape=jax.ShapeDtypeStruct((1, 1), jnp.float32),
    )(sums, cent)


def kernel(fushed_features, input_data, centroids, Wc1a, bc1a, Wc1b, bc1b,
           Wc2, bc2, Wi1a, bi1a, Wi1b, bi1b, Wi2, bi2, Wg, bg, adj):
    fushed = fushed_features.reshape(R, F)
    inp = input_data[:, 0].reshape(R, SX)
    wgt = jnp.transpose(Wg, (1, 0, 2)).reshape(SX, CSO)
    bgt = bg.reshape(1, CSO)
    r2 = lambda v: v.reshape(1, -1)

    hs = _stage1(fushed, inp, centroids,
                 Wc1a, r2(bc1a), Wc1b, r2(bc1b), Wc2, r2(bc2),
                 Wi1a, r2(bi1a), Wi1b, r2(bi1b), Wi2, r2(bi2), wgt, bgt)

    adjg = adj + (jnp.arange(B, dtype=jnp.int32) * N)[:, None, None]
    adjt = jnp.transpose(adjg, (2, 0, 1)).reshape(K, R)
    adjt = jnp.pad(adjt, ((0, 0), (0, RP - R))).reshape(K, NW, NCH, CH)

    nd = _stage2(hs, adjt)
    upd, sums = _stage3(nd)
    loss = _stage4(sums.reshape(C, SO), centroids)
    return upd.reshape(B, N, C, SO), loss[0, 0]


# speedup vs baseline: 1.0126x; 1.0126x over previous
import functools

import jax
import jax.numpy as jnp
from jax import lax
from jax.experimental import pallas as pl
from jax.experimental.pallas import tpu as pltpu
from jax.experimental.pallas import tpu_sc as plsc

B, N, K, SX, F, MID, DOUT, C, SO = 4, 10000, 16, 12, 32, 16, 16, 8, 32
MARGIN = 0.5
UPDATE_RATE = 0.01

R = B * N
CSO = C * SO
W = CSO + 2 * C
NCORES, NSUB = 2, 16
NW = NCORES * NSUB
CH = 256
NCH = 5
CHB = CH // 128
PER_TILE = CH * NCH
RP = NW * PER_TILE
BLK = 2000
NBLK = R // BLK


def _relu(x):
    return jnp.maximum(x, 0.0)


def _dot(a, b):
    return jnp.dot(a, b, preferred_element_type=jnp.float32)


def _expand_mat(rows, cols, group):
    r = lax.broadcasted_iota(jnp.int32, (rows, cols), 0)
    j = lax.broadcasted_iota(jnp.int32, (rows, cols), 1)
    return (j // group == r).astype(jnp.float32)


def _s1_body(fushed_ref, inp_ref, cent_ref, wc1a, bc1a, wc1b, bc1b, wc2, bc2,
             wi1a, bi1a, wi1b, bi1b, wi2, bi2, wgt, bgt, out_ref):
    x = fushed_ref[...]
    fused = _relu(_dot(_relu(_dot(x, wi1a[...]) + bi1a[...]), wi1b[...])
                  + bi1b[...]) + _relu(_dot(x, wi2[...]) + bi2[...])
    cent = cent_ref[...]
    cf = _relu(_dot(_relu(_dot(cent, wc1a[...]) + bc1a[...]), wc1b[...])
               + bc1b[...]) + _relu(_dot(cent, wc2[...]) + bc2[...])
    xn = jnp.sum(fused * fused, axis=1, keepdims=True)
    cn = lax.dot_general(jnp.ones((1, DOUT), jnp.float32), cf * cf,
                         (((1,), (1,)), ((), ())))
    g = lax.dot_general(fused, cf, (((1,), (1,)), ((), ())))
    dist = jnp.sqrt(jnp.maximum(xn + cn - 2.0 * g, 1e-30))
    m = jnp.max(dist, axis=1, keepdims=True)
    e = jnp.exp(dist - m)
    simi = e / jnp.sum(e, axis=1, keepdims=True)
    t = _relu(_dot(inp_ref[...], wgt[...]) + bgt[...])
    h = t * _dot(simi, _expand_mat(C, CSO, SO))
    out_ref[...] = jnp.concatenate(
        [h, simi, jnp.zeros((BLK, C), jnp.float32)], axis=1)


def _stage1(fushed, inp, cent, wc1a, bc1a, wc1b, bc1b, wc2, bc2,
            wi1a, bi1a, wi1b, bi1b, wi2, bi2, wgt, bgt):
    row_spec = lambda width: pl.BlockSpec((BLK, width), lambda i: (i, 0))
    rep = lambda shape: pl.BlockSpec(shape, lambda i: (0,) * len(shape))
    return pl.pallas_call(
        _s1_body,
        grid=(NBLK,),
        in_specs=[row_spec(F), row_spec(SX), rep((C, F)),
                  rep((F, MID)), rep((1, MID)), rep((MID, DOUT)), rep((1, DOUT)),
                  rep((F, DOUT)), rep((1, DOUT)),
                  rep((F, MID)), rep((1, MID)), rep((MID, DOUT)), rep((1, DOUT)),
                  rep((F, DOUT)), rep((1, DOUT)),
                  rep((SX, CSO)), rep((1, CSO))],
        out_specs=row_spec(W),
        out_shape=jax.ShapeDtypeStruct((R, W), jnp.float32),
    )(fushed, inp, cent, wc1a, bc1a, wc1b, bc1b, wc2, bc2,
      wi1a, bi1a, wi1b, bi1b, wi2, bi2, wgt, bgt)


def _s2_body(hs_hbm, adjt_hbm, out_hbm, idx_v, acc_v, sem0, sem1):
    wid = lax.axis_index("s") * NCORES + lax.axis_index("c")
    pltpu.sync_copy(adjt_hbm.at[:, wid], idx_v)

    def chunk(j, carry):
        pltpu.async_copy(hs_hbm.at[idx_v.at[0, j]], acc_v, sem0).wait()
        descs = [pltpu.async_copy(hs_hbm.at[idx_v.at[k, j]], acc_v, sem1,
                                  add=True) for k in range(1, K)]
        for d in descs:
            d.wait()
        pltpu.sync_copy(acc_v, out_hbm.at[pl.ds(wid * PER_TILE + j * CH, CH)])
        return carry

    lax.fori_loop(0, NCH, chunk, 0)


_stage2 = functools.partial(
    pl.kernel,
    out_type=jax.ShapeDtypeStruct((RP, W), jnp.float32),
    mesh=plsc.VectorSubcoreMesh(core_axis_name="c", subcore_axis_name="s",
                                num_cores=NCORES, num_subcores=NSUB),
    scratch_types=[pltpu.VMEM((K, NCH, CH), jnp.int32),
                   pltpu.VMEM((CH, W), jnp.float32),
                   pltpu.SemaphoreType.DMA,
                   pltpu.SemaphoreType.DMA],
    compiler_params=pltpu.CompilerParams(use_tc_tiling_on_sc=False),
)(_s2_body)


def _s3_body(nd_ref, upd_ref, sum_ref):
    nd = nd_ref[...]
    numer = nd[:, :CSO]
    denom = nd[:, CSO:CSO + C]
    upd = numer / _dot(denom, _expand_mat(C, CSO, SO))
    upd_ref[...] = upd
    part = jnp.sum(upd, axis=0, keepdims=True)

    @pl.when(pl.program_id(0) == 0)
    def _():
        sum_ref[...] = part

    @pl.when(pl.program_id(0) > 0)
    def _():
        sum_ref[...] = sum_ref[...] + part


def _stage3(nd):
    return pl.pallas_call(
        _s3_body,
        grid=(NBLK,),
        in_specs=[pl.BlockSpec((BLK, W), lambda i: (i, 0))],
        out_specs=[pl.BlockSpec((BLK, CSO), lambda i: (i, 0)),
                   pl.BlockSpec((1, CSO), lambda i: (0, 0))],
        out_shape=[jax.ShapeDtypeStruct((R, CSO), jnp.float32),
                   jax.ShapeDtypeStruct((1, CSO), jnp.float32)],
    )(nd)


def _s4_body(sum_ref, cent_ref, out_ref):
    mean = sum_ref[...] * (1.0 / R)
    nc = (1.0 - UPDATE_RATE) * cent_ref[...] + UPDATE_RATE * mean
    sq = nc * nc
    ones = jnp.ones((1, SO), jnp.float32)
    ncol = lax.dot_general(ones, sq, (((1,), (1,)), ((), ())))
    nrow = lax.dot_general(sq, ones, (((1,), (1,)), ((), ())))
    g = lax.dot_general(nc, nc, (((1,), (1,)), ((), ())))
    dist = jnp.sqrt(jnp.maximum(nrow + ncol - 2.0 * g, 1e-30))
    i = lax.broadcasted_iota(jnp.int32, (C, C), 0)
    j = lax.broadcasted_iota(jnp.int32, (C, C), 1)
    target = jnp.where(i == j, 0.0, MARGIN)
    out_ref[...] = jnp.sum(jnp.maximum(target - dist, 0.0) ** 2,
                           keepdims=True)


def _stage4(sums, cent):
    return pl.pallas_call(
        _s4_body,
        in_specs=[pl.BlockSpec((C, SO), lambda: (0, 0)),
                  pl.BlockSpec((C, SO), lambda: (0, 0))],
        out_specs=pl.BlockSpec((1, 1), lambda: (0, 0)),
        out_shape=jax.ShapeDtypeStruct((1, 1), jnp.float32),
    )(sums, cent)


def kernel(fushed_features, input_data, centroids, Wc1a, bc1a, Wc1b, bc1b,
           Wc2, bc2, Wi1a, bi1a, Wi1b, bi1b, Wi2, bi2, Wg, bg, adj):
    fushed = fushed_features.reshape(R, F)
    inp = input_data[:, 0].reshape(R, SX)
    wgt = jnp.transpose(Wg, (1, 0, 2)).reshape(SX, CSO)
    bgt = bg.reshape(1, CSO)
    r2 = lambda v: v.reshape(1, -1)

    hs = _stage1(fushed, inp, centroids,
                 Wc1a, r2(bc1a), Wc1b, r2(bc1b), Wc2, r2(bc2),
                 Wi1a, r2(bi1a), Wi1b, r2(bi1b), Wi2, r2(bi2), wgt, bgt)

    adjg = adj + (jnp.arange(B, dtype=jnp.int32) * N)[:, None, None]
    adjt = jnp.transpose(adjg, (2, 0, 1)).reshape(K, R)
    adjt = jnp.pad(adjt, ((0, 0), (0, RP - R))).reshape(K, NW, NCH, CH)

    nd = _stage2(hs, adjt)
    upd, sums = _stage3(nd)
    loss = _stage4(sums.reshape(C, SO), centroids)
    return upd.reshape(B, N, C, SO), loss[0, 0]


# speedup vs baseline: 1.0592x; 1.0460x over previous
import functools

import jax
import jax.numpy as jnp
from jax import lax
from jax.experimental import pallas as pl
from jax.experimental.pallas import tpu as pltpu
from jax.experimental.pallas import tpu_sc as plsc

B, N, K, SX, F, MID, DOUT, C, SO = 4, 10000, 16, 12, 32, 16, 16, 8, 32
MARGIN = 0.5
UPDATE_RATE = 0.01

R = B * N
CSO = C * SO
W = CSO + 2 * C
NCORES, NSUB = 2, 16
NW = NCORES * NSUB
CH = 256
NCH0 = 8
NCH1 = 2
NCHMAX = max(NCH0, NCH1)
TOTAL_CHUNKS = NSUB * (NCH0 + NCH1)
RP = TOTAL_CHUNKS * CH
BLK = 2000
NBLK = R // BLK


def _relu(x):
    return jnp.maximum(x, 0.0)


def _dot(a, b):
    return jnp.dot(a, b, preferred_element_type=jnp.float32)


def _expand_mat(rows, cols, group):
    r = lax.broadcasted_iota(jnp.int32, (rows, cols), 0)
    j = lax.broadcasted_iota(jnp.int32, (rows, cols), 1)
    return (j // group == r).astype(jnp.float32)


def _s1_body(fushed_ref, inp_ref, cent_ref, wc1a, bc1a, wc1b, bc1b, wc2, bc2,
             wi1a, bi1a, wi1b, bi1b, wi2, bi2, wgt, bgt, out_ref):
    x = fushed_ref[...]
    fused = _relu(_dot(_relu(_dot(x, wi1a[...]) + bi1a[...]), wi1b[...])
                  + bi1b[...]) + _relu(_dot(x, wi2[...]) + bi2[...])
    cent = cent_ref[...]
    cf = _relu(_dot(_relu(_dot(cent, wc1a[...]) + bc1a[...]), wc1b[...])
               + bc1b[...]) + _relu(_dot(cent, wc2[...]) + bc2[...])
    xn = jnp.sum(fused * fused, axis=1, keepdims=True)
    cn = lax.dot_general(jnp.ones((1, DOUT), jnp.float32), cf * cf,
                         (((1,), (1,)), ((), ())))
    g = lax.dot_general(fused, cf, (((1,), (1,)), ((), ())))
    dist = jnp.sqrt(jnp.maximum(xn + cn - 2.0 * g, 1e-30))
    m = jnp.max(dist, axis=1, keepdims=True)
    e = jnp.exp(dist - m)
    simi = e / jnp.sum(e, axis=1, keepdims=True)
    t = _relu(_dot(inp_ref[...], wgt[...]) + bgt[...])
    h = t * _dot(simi, _expand_mat(C, CSO, SO))
    out_ref[...] = jnp.concatenate(
        [h, simi, jnp.zeros((BLK, C), jnp.float32)], axis=1)


def _stage1(fushed, inp, cent, wc1a, bc1a, wc1b, bc1b, wc2, bc2,
            wi1a, bi1a, wi1b, bi1b, wi2, bi2, wgt, bgt):
    row_spec = lambda width: pl.BlockSpec((BLK, width), lambda i: (i, 0))
    rep = lambda shape: pl.BlockSpec(shape, lambda i: (0,) * len(shape))
    return pl.pallas_call(
        _s1_body,
        grid=(NBLK,),
        in_specs=[row_spec(F), row_spec(SX), rep((C, F)),
                  rep((F, MID)), rep((1, MID)), rep((MID, DOUT)), rep((1, DOUT)),
                  rep((F, DOUT)), rep((1, DOUT)),
                  rep((F, MID)), rep((1, MID)), rep((MID, DOUT)), rep((1, DOUT)),
                  rep((F, DOUT)), rep((1, DOUT)),
                  rep((SX, CSO)), rep((1, CSO))],
        out_specs=row_spec(W),
        out_shape=jax.ShapeDtypeStruct((R, W), jnp.float32),
    )(fushed, inp, cent, wc1a, bc1a, wc1b, bc1b, wc2, bc2,
      wi1a, bi1a, wi1b, bi1b, wi2, bi2, wgt, bgt)


def _s2_body(hs_hbm, adjt_hbm, out_hbm, idx_v, acc_v, sem0, sem1):
    c = lax.axis_index("c")
    s = lax.axis_index("s")
    nch = jnp.where(c == 0, NCH0, NCH1)
    start = jnp.where(c == 0, s * NCH0, NSUB * NCH0 + s * NCH1)

    @pl.when(c == 0)
    def _():
        pltpu.sync_copy(adjt_hbm.at[:, pl.ds(s * NCH0, NCH0)], idx_v)

    @pl.when(c == 1)
    def _():
        pltpu.sync_copy(adjt_hbm.at[:, pl.ds(NSUB * NCH0 + s * NCH1, NCH1)],
                        idx_v.at[:, pl.ds(0, NCH1)])

    def chunk(j, carry):
        pltpu.async_copy(hs_hbm.at[idx_v.at[0, j]], acc_v, sem0).wait()
        descs = [pltpu.async_copy(hs_hbm.at[idx_v.at[k, j]], acc_v, sem1,
                                  add=True) for k in range(1, K)]
        for d in descs:
            d.wait()
        pltpu.sync_copy(acc_v, out_hbm.at[pl.ds((start + j) * CH, CH)])
        return carry

    lax.fori_loop(0, nch, chunk, 0)


_stage2 = functools.partial(
    pl.kernel,
    out_type=jax.ShapeDtypeStruct((RP, W), jnp.float32),
    mesh=plsc.VectorSubcoreMesh(core_axis_name="c", subcore_axis_name="s",
                                num_cores=NCORES, num_subcores=NSUB),
    scratch_types=[pltpu.VMEM((K, NCHMAX, CH), jnp.int32),
                   pltpu.VMEM((CH, W), jnp.float32),
                   pltpu.SemaphoreType.DMA,
                   pltpu.SemaphoreType.DMA],
    compiler_params=pltpu.CompilerParams(use_tc_tiling_on_sc=False),
)(_s2_body)


def _s3_body(nd_ref, upd_ref, sum_ref):
    nd = nd_ref[...]
    numer = nd[:, :CSO]
    denom = nd[:, CSO:CSO + C]
    upd = numer / _dot(denom, _expand_mat(C, CSO, SO))
    upd_ref[...] = upd
    part = jnp.sum(upd, axis=0, keepdims=True)

    @pl.when(pl.program_id(0) == 0)
    def _():
        sum_ref[...] = part

    @pl.when(pl.program_id(0) > 0)
    def _():
        sum_ref[...] = sum_ref[...] + part


def _stage3(nd):
    return pl.pallas_call(
        _s3_body,
        grid=(NBLK,),
        in_specs=[pl.BlockSpec((BLK, W), lambda i: (i, 0))],
        out_specs=[pl.BlockSpec((BLK, CSO), lambda i: (i, 0)),
                   pl.BlockSpec((1, CSO), lambda i: (0, 0))],
        out_shape=[jax.ShapeDtypeStruct((R, CSO), jnp.float32),
                   jax.ShapeDtypeStruct((1, CSO), jnp.float32)],
    )(nd)


def _s4_body(sum_ref, cent_ref, out_ref):
    mean = sum_ref[...] * (1.0 / R)
    nc = (1.0 - UPDATE_RATE) * cent_ref[...] + UPDATE_RATE * mean
    sq = nc * nc
    ones = jnp.ones((1, SO), jnp.float32)
    ncol = lax.dot_general(ones, sq, (((1,), (1,)), ((), ())))
    nrow = lax.dot_general(sq, ones, (((1,), (1,)), ((), ())))
    g = lax.dot_general(nc, nc, (((1,), (1,)), ((), ())))
    dist = jnp.sqrt(jnp.maximum(nrow + ncol - 2.0 * g, 1e-30))
    i = lax.broadcasted_iota(jnp.int32, (C, C), 0)
    j = lax.broadcasted_iota(jnp.int32, (C, C), 1)
    target = jnp.where(i == j, 0.0, MARGIN)
    out_ref[...] = jnp.sum(jnp.maximum(target - dist, 0.0) ** 2,
                           keepdims=True)


def _stage4(sums, cent):
    return pl.pallas_call(
        _s4_body,
        in_specs=[pl.BlockSpec((C, SO), lambda: (0, 0)),
                  pl.BlockSpec((C, SO), lambda: (0, 0))],
        out_specs=pl.BlockSpec((1, 1), lambda: (0, 0)),
        out_shape=jax.ShapeDtypeStruct((1, 1), jnp.float32),
    )(sums, cent)


def kernel(fushed_features, input_data, centroids, Wc1a, bc1a, Wc1b, bc1b,
           Wc2, bc2, Wi1a, bi1a, Wi1b, bi1b, Wi2, bi2, Wg, bg, adj):
    fushed = fushed_features.reshape(R, F)
    inp = input_data[:, 0].reshape(R, SX)
    wgt = jnp.transpose(Wg, (1, 0, 2)).reshape(SX, CSO)
    bgt = bg.reshape(1, CSO)
    r2 = lambda v: v.reshape(1, -1)

    hs = _stage1(fushed, inp, centroids,
                 Wc1a, r2(bc1a), Wc1b, r2(bc1b), Wc2, r2(bc2),
                 Wi1a, r2(bi1a), Wi1b, r2(bi1b), Wi2, r2(bi2), wgt, bgt)

    adjg = adj + (jnp.arange(B, dtype=jnp.int32) * N)[:, None, None]
    adjt = jnp.transpose(adjg, (2, 0, 1)).reshape(K, R)
    adjt = jnp.pad(adjt, ((0, 0), (0, RP - R))).reshape(K, TOTAL_CHUNKS, CH)

    nd = _stage2(hs, adjt)
    upd, sums = _stage3(nd)
    loss = _stage4(sums.reshape(C, SO), centroids)
    return upd.reshape(B, N, C, SO), loss[0, 0]


# speedup vs baseline: 2.0253x; 1.9121x over previous
import functools

import jax
import jax.numpy as jnp
from jax import lax
from jax.experimental import pallas as pl
from jax.experimental.pallas import tpu as pltpu
from jax.experimental.pallas import tpu_sc as plsc

B, N, K, SX, F, MID, DOUT, C, SO = 4, 10000, 16, 12, 32, 16, 16, 8, 32
MARGIN = 0.5
UPDATE_RATE = 0.01

R = B * N
RK = R * K
CSO = C * SO
TW = 32
NCORES, NSUB = 2, 16
NW = NCORES * NSUB
PER_TILE = RK // NW
CH = 500
NCH = PER_TILE // CH
NBUF = 4
BLK = 2000
NBLK = R // BLK
BLK3 = 400
NBLK3 = R // BLK3


def _relu(x):
    return jnp.maximum(x, 0.0)


def _dot(a, b):
    return jnp.dot(a, b, preferred_element_type=jnp.float32)


def _expand_mat():
    r = lax.broadcasted_iota(jnp.int32, (C, CSO), 0)
    j = lax.broadcasted_iota(jnp.int32, (C, CSO), 1)
    return (j // SO == r).astype(jnp.float32)


def _s1_body(fushed_ref, inp_ref, cent_ref, wc1a, bc1a, wc1b, bc1b, wc2, bc2,
             wi1a, bi1a, wi1b, bi1b, wi2, bi2, out_ref):
    x = fushed_ref[...]
    fused = _relu(_dot(_relu(_dot(x, wi1a[...]) + bi1a[...]), wi1b[...])
                  + bi1b[...]) + _relu(_dot(x, wi2[...]) + bi2[...])
    cent = cent_ref[...]
    cf = _relu(_dot(_relu(_dot(cent, wc1a[...]) + bc1a[...]), wc1b[...])
               + bc1b[...]) + _relu(_dot(cent, wc2[...]) + bc2[...])
    xn = jnp.sum(fused * fused, axis=1, keepdims=True)
    cn = lax.dot_general(jnp.ones((1, DOUT), jnp.float32), cf * cf,
                         (((1,), (1,)), ((), ())))
    g = lax.dot_general(fused, cf, (((1,), (1,)), ((), ())))
    dist = jnp.sqrt(jnp.maximum(xn + cn - 2.0 * g, 1e-30))
    m = jnp.max(dist, axis=1, keepdims=True)
    e = jnp.exp(dist - m)
    simi = e / jnp.sum(e, axis=1, keepdims=True)
    out_ref[...] = jnp.concatenate(
        [inp_ref[...], simi, jnp.zeros((BLK, TW - SX - C), jnp.float32)],
        axis=1)


def _stage1(fushed, inp, cent, wc1a, bc1a, wc1b, bc1b, wc2, bc2,
            wi1a, bi1a, wi1b, bi1b, wi2, bi2):
    row_spec = lambda width: pl.BlockSpec((BLK, width), lambda i: (i, 0))
    rep = lambda shape: pl.BlockSpec(shape, lambda i: (0,) * len(shape))
    return pl.pallas_call(
        _s1_body,
        grid=(NBLK,),
        in_specs=[row_spec(F), row_spec(SX), rep((C, F)),
                  rep((F, MID)), rep((1, MID)), rep((MID, DOUT)), rep((1, DOUT)),
                  rep((F, DOUT)), rep((1, DOUT)),
                  rep((F, MID)), rep((1, MID)), rep((MID, DOUT)), rep((1, DOUT)),
                  rep((F, DOUT)), rep((1, DOUT))],
        out_specs=row_spec(TW),
        out_shape=jax.ShapeDtypeStruct((R, TW), jnp.float32),
    )(fushed, inp, cent, wc1a, bc1a, wc1b, bc1b, wc2, bc2,
      wi1a, bi1a, wi1b, bi1b, wi2, bi2)


def _s2_body(t1_hbm, adjg_hbm, out_hbm, idx_v, st_v, semg):
    wid = lax.axis_index("s") * NCORES + lax.axis_index("c")
    base = wid * PER_TILE
    pltpu.sync_copy(adjg_hbm.at[wid], idx_v)

    def group(p, carry):
        descs = [pltpu.async_copy(t1_hbm.at[idx_v.at[p * NBUF + q]],
                                  st_v.at[q], semg) for q in range(NBUF)]
        for q in range(NBUF):
            descs[q].wait()
            pltpu.sync_copy(
                st_v.at[q],
                out_hbm.at[pl.ds(base + (p * NBUF + q) * CH, CH)])
        return carry

    lax.fori_loop(0, NCH // NBUF, group, 0)


_stage2 = functools.partial(
    pl.kernel,
    out_type=jax.ShapeDtypeStruct((RK, TW), jnp.float32),
    mesh=plsc.VectorSubcoreMesh(core_axis_name="c", subcore_axis_name="s",
                                num_cores=NCORES, num_subcores=NSUB),
    scratch_types=[pltpu.VMEM((NCH, CH), jnp.int32),
                   pltpu.VMEM((NBUF, CH, TW), jnp.float32),
                   pltpu.SemaphoreType.DMA],
    compiler_params=pltpu.CompilerParams(use_tc_tiling_on_sc=False),
)(_s2_body)


def _s3_body(g_ref, wgt, bgt, upd_ref, sum_ref):
    g = g_ref[...]
    xi = g[:, :SX]
    s = g[:, SX:SX + C]
    t = _relu(_dot(xi, wgt[...]) + bgt[...])
    e = _expand_mat()
    h = t * _dot(s, e)
    numer = jnp.sum(h.reshape(BLK3, K, CSO), axis=1)
    denom = jnp.sum(s.reshape(BLK3, K, C), axis=1)
    upd = numer / _dot(denom, e)
    upd_ref[...] = upd
    part = jnp.sum(upd, axis=0, keepdims=True)

    @pl.when(pl.program_id(0) == 0)
    def _():
        sum_ref[...] = part

    @pl.when(pl.program_id(0) > 0)
    def _():
        sum_ref[...] = sum_ref[...] + part


def _stage3(gathered, wgt, bgt):
    rep = lambda shape: pl.BlockSpec(shape, lambda i: (0,) * len(shape))
    return pl.pallas_call(
        _s3_body,
        grid=(NBLK3,),
        in_specs=[pl.BlockSpec((BLK3 * K, TW), lambda i: (i, 0)),
                  rep((SX, CSO)), rep((1, CSO))],
        out_specs=[pl.BlockSpec((BLK3, CSO), lambda i: (i, 0)),
                   pl.BlockSpec((1, CSO), lambda i: (0, 0))],
        out_shape=[jax.ShapeDtypeStruct((R, CSO), jnp.float32),
                   jax.ShapeDtypeStruct((1, CSO), jnp.float32)],
    )(gathered, wgt, bgt)


def _s4_body(sum_ref, cent_ref, out_ref):
    mean = sum_ref[...] * (1.0 / R)
    nc = (1.0 - UPDATE_RATE) * cent_ref[...] + UPDATE_RATE * mean
    sq = nc * nc
    ones = jnp.ones((1, SO), jnp.float32)
    ncol = lax.dot_general(ones, sq, (((1,), (1,)), ((), ())))
    nrow = lax.dot_general(sq, ones, (((1,), (1,)), ((), ())))
    g = lax.dot_general(nc, nc, (((1,), (1,)), ((), ())))
    dist = jnp.sqrt(jnp.maximum(nrow + ncol - 2.0 * g, 1e-30))
    i = lax.broadcasted_iota(jnp.int32, (C, C), 0)
    j = lax.broadcasted_iota(jnp.int32, (C, C), 1)
    target = jnp.where(i == j, 0.0, MARGIN)
    out_ref[...] = jnp.sum(jnp.maximum(target - dist, 0.0) ** 2,
                           keepdims=True)


def _stage4(sums, cent):
    return pl.pallas_call(
        _s4_body,
        in_specs=[pl.BlockSpec((C, SO), lambda: (0, 0)),
                  pl.BlockSpec((C, SO), lambda: (0, 0))],
        out_specs=pl.BlockSpec((1, 1), lambda: (0, 0)),
        out_shape=jax.ShapeDtypeStruct((1, 1), jnp.float32),
    )(sums, cent)


def kernel(fushed_features, input_data, centroids, Wc1a, bc1a, Wc1b, bc1b,
           Wc2, bc2, Wi1a, bi1a, Wi1b, bi1b, Wi2, bi2, Wg, bg, adj):
    fushed = fushed_features.reshape(R, F)
    inp = input_data[:, 0].reshape(R, SX)
    wgt = jnp.transpose(Wg, (1, 0, 2)).reshape(SX, CSO)
    bgt = bg.reshape(1, CSO)
    r2 = lambda v: v.reshape(1, -1)

    t1 = _stage1(fushed, inp, centroids,
                 Wc1a, r2(bc1a), Wc1b, r2(bc1b), Wc2, r2(bc2),
                 Wi1a, r2(bi1a), Wi1b, r2(bi1b), Wi2, r2(bi2))

    adjg = adj + (jnp.arange(B, dtype=jnp.int32) * N)[:, None, None]
    adjg = adjg.reshape(NW, NCH, CH)

    gathered = _stage2(t1, adjg)
    upd, sums = _stage3(gathered, wgt, bgt)
    loss = _stage4(sums.reshape(C, SO), centroids)
    return upd.reshape(B, N, C, SO), loss[0, 0]


# speedup vs baseline: 2.0515x; 1.0129x over previous
import functools

import jax
import jax.numpy as jnp
from jax import lax
from jax.experimental import pallas as pl
from jax.experimental.pallas import tpu as pltpu
from jax.experimental.pallas import tpu_sc as plsc

B, N, K, SX, F, MID, DOUT, C, SO = 4, 10000, 16, 12, 32, 16, 16, 8, 32
MARGIN = 0.5
UPDATE_RATE = 0.01

R = B * N
RK = R * K
CSO = C * SO
TW = 32
NCORES, NSUB = 2, 16
NW = NCORES * NSUB
PER_TILE = RK // NW
CH = 500
NCH = PER_TILE // CH
NBUF = 4
BLK = 2000
NBLK = R // BLK
BLK3 = 400
NBLK3 = R // BLK3
PACK = 128 // TW
RK4 = RK // PACK
CH4 = CH // PACK


def _relu(x):
    return jnp.maximum(x, 0.0)


def _dot(a, b):
    return jnp.dot(a, b, preferred_element_type=jnp.float32)


def _expand_mat():
    r = lax.broadcasted_iota(jnp.int32, (C, CSO), 0)
    j = lax.broadcasted_iota(jnp.int32, (C, CSO), 1)
    return (j // SO == r).astype(jnp.float32)


def _s1_body(fushed_ref, inp_ref, cent_ref, wc1a, bc1a, wc1b, bc1b, wc2, bc2,
             wi1a, bi1a, wi1b, bi1b, wi2, bi2, out_ref):
    x = fushed_ref[...]
    fused = _relu(_dot(_relu(_dot(x, wi1a[...]) + bi1a[...]), wi1b[...])
                  + bi1b[...]) + _relu(_dot(x, wi2[...]) + bi2[...])
    cent = cent_ref[...]
    cf = _relu(_dot(_relu(_dot(cent, wc1a[...]) + bc1a[...]), wc1b[...])
               + bc1b[...]) + _relu(_dot(cent, wc2[...]) + bc2[...])
    xn = jnp.sum(fused * fused, axis=1, keepdims=True)
    cn = lax.dot_general(jnp.ones((1, DOUT), jnp.float32), cf * cf,
                         (((1,), (1,)), ((), ())))
    g = lax.dot_general(fused, cf, (((1,), (1,)), ((), ())))
    dist = jnp.sqrt(jnp.maximum(xn + cn - 2.0 * g, 1e-30))
    m = jnp.max(dist, axis=1, keepdims=True)
    e = jnp.exp(dist - m)
    simi = e / jnp.sum(e, axis=1, keepdims=True)
    out_ref[...] = jnp.concatenate(
        [inp_ref[...], simi, jnp.zeros((BLK, TW - SX - C), jnp.float32)],
        axis=1)


def _stage1(fushed, inp, cent, wc1a, bc1a, wc1b, bc1b, wc2, bc2,
            wi1a, bi1a, wi1b, bi1b, wi2, bi2):
    row_spec = lambda width: pl.BlockSpec((BLK, width), lambda i: (i, 0))
    rep = lambda shape: pl.BlockSpec(shape, lambda i: (0,) * len(shape))
    return pl.pallas_call(
        _s1_body,
        grid=(NBLK,),
        in_specs=[row_spec(F), row_spec(SX), rep((C, F)),
                  rep((F, MID)), rep((1, MID)), rep((MID, DOUT)), rep((1, DOUT)),
                  rep((F, DOUT)), rep((1, DOUT)),
                  rep((F, MID)), rep((1, MID)), rep((MID, DOUT)), rep((1, DOUT)),
                  rep((F, DOUT)), rep((1, DOUT))],
        out_specs=row_spec(TW),
        out_shape=jax.ShapeDtypeStruct((R, TW), jnp.float32),
    )(fushed, inp, cent, wc1a, bc1a, wc1b, bc1b, wc2, bc2,
      wi1a, bi1a, wi1b, bi1b, wi2, bi2)


def _s2_body(t1_hbm, adjg_hbm, out_hbm, idx_v, st_v, semg):
    wid = lax.axis_index("s") * NCORES + lax.axis_index("c")
    base4 = wid * (PER_TILE // PACK)
    pltpu.sync_copy(adjg_hbm.at[wid], idx_v)

    def chunk(j, carry):
        descs = [pltpu.async_copy(t1_hbm.at[idx_v.at[j, g]],
                                  st_v.at[g], semg) for g in range(PACK)]
        for g in range(PACK):
            descs[g].wait()
            pltpu.sync_copy(
                st_v.at[g],
                out_hbm.at[pl.ds(base4 + j * CH4, CH4), pl.ds(g * TW, TW)])
        return carry

    lax.fori_loop(0, NCH, chunk, 0)


_stage2 = functools.partial(
    pl.kernel,
    out_type=jax.ShapeDtypeStruct((RK4, 128), jnp.float32),
    mesh=plsc.VectorSubcoreMesh(core_axis_name="c", subcore_axis_name="s",
                                num_cores=NCORES, num_subcores=NSUB),
    scratch_types=[pltpu.VMEM((NCH, PACK, CH4), jnp.int32),
                   pltpu.VMEM((PACK, CH4, TW), jnp.float32),
                   pltpu.SemaphoreType.DMA],
    compiler_params=pltpu.CompilerParams(use_tc_tiling_on_sc=False),
)(_s2_body)


def _s3_body(g_ref, wgt, bgt, upd_ref, sum_ref):
    g = g_ref[...]
    e = _expand_mat()
    hsum = None
    ssum = None
    for q in range(PACK):
        xi = g[:, q * TW:q * TW + SX]
        s = g[:, q * TW + SX:q * TW + SX + C]
        t = _relu(_dot(xi, wgt[...]) + bgt[...])
        h = t * _dot(s, e)
        hsum = h if hsum is None else hsum + h
        ssum = s if ssum is None else ssum + s
    kp = K // PACK
    numer = jnp.sum(hsum.reshape(BLK3, kp, CSO), axis=1)
    denom = jnp.sum(ssum.reshape(BLK3, kp, C), axis=1)
    upd = numer / _dot(denom, e)
    upd_ref[...] = upd
    part = jnp.sum(upd, axis=0, keepdims=True)

    @pl.when(pl.program_id(0) == 0)
    def _():
        sum_ref[...] = part

    @pl.when(pl.program_id(0) > 0)
    def _():
        sum_ref[...] = sum_ref[...] + part


def _stage3(gathered, wgt, bgt):
    rep = lambda shape: pl.BlockSpec(shape, lambda i: (0,) * len(shape))
    return pl.pallas_call(
        _s3_body,
        grid=(NBLK3,),
        in_specs=[pl.BlockSpec((BLK3 * K // PACK, 128), lambda i: (i, 0)),
                  rep((SX, CSO)), rep((1, CSO))],
        out_specs=[pl.BlockSpec((BLK3, CSO), lambda i: (i, 0)),
                   pl.BlockSpec((1, CSO), lambda i: (0, 0))],
        out_shape=[jax.ShapeDtypeStruct((R, CSO), jnp.float32),
                   jax.ShapeDtypeStruct((1, CSO), jnp.float32)],
    )(gathered, wgt, bgt)


def _s4_body(sum_ref, cent_ref, out_ref):
    mean = sum_ref[...] * (1.0 / R)
    nc = (1.0 - UPDATE_RATE) * cent_ref[...] + UPDATE_RATE * mean
    sq = nc * nc
    ones = jnp.ones((1, SO), jnp.float32)
    ncol = lax.dot_general(ones, sq, (((1,), (1,)), ((), ())))
    nrow = lax.dot_general(sq, ones, (((1,), (1,)), ((), ())))
    g = lax.dot_general(nc, nc, (((1,), (1,)), ((), ())))
    dist = jnp.sqrt(jnp.maximum(nrow + ncol - 2.0 * g, 1e-30))
    i = lax.broadcasted_iota(jnp.int32, (C, C), 0)
    j = lax.broadcasted_iota(jnp.int32, (C, C), 1)
    target = jnp.where(i == j, 0.0, MARGIN)
    out_ref[...] = jnp.sum(jnp.maximum(target - dist, 0.0) ** 2,
                           keepdims=True)


def _stage4(sums, cent):
    return pl.pallas_call(
        _s4_body,
        in_specs=[pl.BlockSpec((C, SO), lambda: (0, 0)),
                  pl.BlockSpec((C, SO), lambda: (0, 0))],
        out_specs=pl.BlockSpec((1, 1), lambda: (0, 0)),
        out_shape=jax.ShapeDtypeStruct((1, 1), jnp.float32),
    )(sums, cent)


def kernel(fushed_features, input_data, centroids, Wc1a, bc1a, Wc1b, bc1b,
           Wc2, bc2, Wi1a, bi1a, Wi1b, bi1b, Wi2, bi2, Wg, bg, adj):
    fushed = fushed_features.reshape(R, F)
    inp = input_data[:, 0].reshape(R, SX)
    wgt = jnp.transpose(Wg, (1, 0, 2)).reshape(SX, CSO)
    bgt = bg.reshape(1, CSO)
    r2 = lambda v: v.reshape(1, -1)

    t1 = _stage1(fushed, inp, centroids,
                 Wc1a, r2(bc1a), Wc1b, r2(bc1b), Wc2, r2(bc2),
                 Wi1a, r2(bi1a), Wi1b, r2(bi1b), Wi2, r2(bi2))

    adjg = adj + (jnp.arange(B, dtype=jnp.int32) * N)[:, None, None]
    adjg = adjg.reshape(NW, NCH, CH4, PACK)
    adjg = jnp.transpose(adjg, (0, 1, 3, 2))

    gathered = _stage2(t1, adjg)
    upd, sums = _stage3(gathered, wgt, bgt)
    loss = _stage4(sums.reshape(C, SO), centroids)
    return upd.reshape(B, N, C, SO), loss[0, 0]
